# Initial kernel scaffold; baseline (speedup 1.0000x reference)
import jax
import jax.numpy as jnp
from jax.experimental import pallas as pl


def kernel(z, prototype_vectors):
    raise NotImplementedError("write your pallas kernel here")


# speedup vs baseline: 3.1327x; 3.1327x over previous
import jax
import jax.numpy as jnp
from jax.experimental import pallas as pl
from jax.experimental.pallas import tpu as pltpu

_B = 16384
_P = 1024
_D = 128
_BZ = 1024
_NB = _B // _BZ
_REG1 = 0.05
_REG2 = 0.05


def _loss_body(z_ref, p_ref, out_ref, colmin_ref, rowsum_ref):
    i = pl.program_id(0)
    zb = z_ref[:]
    p = p_ref[:]
    z2 = jnp.sum(zb * zb, axis=1, keepdims=True)
    p2 = jnp.sum(p * p, axis=1)[None, :]
    cross = jax.lax.dot_general(
        zb, p, (((1,), (1,)), ((), ())),
        preferred_element_type=jnp.float32)
    d2 = z2 + p2 - 2.0 * cross
    rowmin = jnp.min(d2, axis=1)
    part = jnp.sum(jnp.sqrt(jnp.maximum(rowmin, 0.0)))
    colmin = jnp.min(d2, axis=0, keepdims=True)

    @pl.when(i == 0)
    def _init():
        rowsum_ref[0] = part
        colmin_ref[:] = colmin

    @pl.when(i > 0)
    def _accum():
        rowsum_ref[0] = rowsum_ref[0] + part
        colmin_ref[:] = jnp.minimum(colmin_ref[:], colmin)

    @pl.when(i == _NB - 1)
    def _finish():
        cm = jnp.sqrt(jnp.maximum(colmin_ref[:], 0.0))
        val = (_REG1 * (rowsum_ref[0] / _B)
               + _REG2 * (jnp.sum(cm) / _P))
        out_ref[...] = jnp.reshape(val, (1, 1))


def kernel(z, prototype_vectors):
    out = pl.pallas_call(
        _loss_body,
        grid=(_NB,),
        in_specs=[
            pl.BlockSpec((_BZ, _D), lambda i: (i, 0)),
            pl.BlockSpec((_P, _D), lambda i: (0, 0)),
        ],
        out_specs=pl.BlockSpec((1, 1), lambda i: (0, 0)),
        out_shape=jax.ShapeDtypeStruct((1, 1), jnp.float32),
        scratch_shapes=[
            pltpu.VMEM((1, _P), jnp.float32),
            pltpu.SMEM((1,), jnp.float32),
        ],
    )(z, prototype_vectors)
    return out[0, 0]


# speedup vs baseline: 3.1726x; 1.0128x over previous
import jax
import jax.numpy as jnp
from jax.experimental import pallas as pl
from jax.experimental.pallas import tpu as pltpu

_B = 16384
_P = 1024
_D = 128
_BZ = 1024
_NB = _B // _BZ
_REG1 = 0.05
_REG2 = 0.05


def _loss_body(z_ref, p_ref, out_ref, colmin_ref, rowsum_ref):
    i = pl.program_id(0)
    zb = z_ref[:]
    p = p_ref[:]
    z2 = jnp.sum(zb * zb, axis=1, keepdims=True)
    p2 = jnp.sum(p * p, axis=1)[None, :]
    cross = jax.lax.dot_general(
        zb.astype(jnp.bfloat16), p.astype(jnp.bfloat16),
        (((1,), (1,)), ((), ())),
        preferred_element_type=jnp.float32)
    d2 = z2 + p2 - 2.0 * cross
    rowmin = jnp.min(d2, axis=1)
    part = jnp.sum(jnp.sqrt(jnp.maximum(rowmin, 0.0)))
    colmin = jnp.min(d2, axis=0, keepdims=True)

    @pl.when(i == 0)
    def _init():
        rowsum_ref[0] = part
        colmin_ref[:] = colmin

    @pl.when(i > 0)
    def _accum():
        rowsum_ref[0] = rowsum_ref[0] + part
        colmin_ref[:] = jnp.minimum(colmin_ref[:], colmin)

    @pl.when(i == _NB - 1)
    def _finish():
        cm = jnp.sqrt(jnp.maximum(colmin_ref[:], 0.0))
        val = (_REG1 * (rowsum_ref[0] / _B)
               + _REG2 * (jnp.sum(cm) / _P))
        out_ref[...] = jnp.reshape(val, (1, 1))


def kernel(z, prototype_vectors):
    out = pl.pallas_call(
        _loss_body,
        grid=(_NB,),
        in_specs=[
            pl.BlockSpec((_BZ, _D), lambda i: (i, 0)),
            pl.BlockSpec((_P, _D), lambda i: (0, 0)),
        ],
        out_specs=pl.BlockSpec((1, 1), lambda i: (0, 0)),
        out_shape=jax.ShapeDtypeStruct((1, 1), jnp.float32),
        scratch_shapes=[
            pltpu.VMEM((1, _P), jnp.float32),
            pltpu.SMEM((1,), jnp.float32),
        ],
    )(z, prototype_vectors)
    return out[0, 0]


# speedup vs baseline: 4.1644x; 1.3126x over previous
import jax
import jax.numpy as jnp
from jax.experimental import pallas as pl
from jax.experimental.pallas import tpu as pltpu

_B = 16384
_P = 1024
_D = 128
_BZ = 1024
_NB = _B // _BZ
_REG1 = 0.05
_REG2 = 0.05


def _loss_body(z_ref, p_ref, out_ref, pm2_ref, p2_ref, colmin_ref,
               rowacc_ref):
    i = pl.program_id(0)

    @pl.when(i == 0)
    def _prep():
        p = p_ref[:]
        pm2_ref[:] = (-2.0 * p).astype(jnp.bfloat16)
        p2_ref[:] = jax.lax.dot_general(
            jnp.ones((1, _D), jnp.float32), p * p,
            (((1,), (1,)), ((), ())),
            preferred_element_type=jnp.float32)

    zb = z_ref[:]
    z2 = jnp.sum(zb * zb, axis=1, keepdims=True)
    cross = jax.lax.dot_general(
        zb.astype(jnp.bfloat16), pm2_ref[:],
        (((1,), (1,)), ((), ())),
        preferred_element_type=jnp.float32)
    t = cross + p2_ref[:]
    rowmin = jnp.min(t, axis=1, keepdims=True)
    rowpart = jnp.sqrt(jnp.maximum(rowmin + z2, 0.0))
    u = t + z2
    colpart = jnp.min(u, axis=0, keepdims=True)

    @pl.when(i == 0)
    def _init():
        rowacc_ref[:] = rowpart
        colmin_ref[:] = colpart

    @pl.when(i > 0)
    def _accum():
        rowacc_ref[:] = rowacc_ref[:] + rowpart
        colmin_ref[:] = jnp.minimum(colmin_ref[:], colpart)

    @pl.when(i == _NB - 1)
    def _finish():
        cm = jnp.sqrt(jnp.maximum(colmin_ref[:], 0.0))
        val = (_REG1 * (jnp.sum(rowacc_ref[:]) / _B)
               + _REG2 * (jnp.sum(cm) / _P))
        out_ref[...] = jnp.reshape(val, (1, 1))


def kernel(z, prototype_vectors):
    out = pl.pallas_call(
        _loss_body,
        grid=(_NB,),
        in_specs=[
            pl.BlockSpec((_BZ, _D), lambda i: (i, 0)),
            pl.BlockSpec((_P, _D), lambda i: (0, 0)),
        ],
        out_specs=pl.BlockSpec((1, 1), lambda i: (0, 0)),
        out_shape=jax.ShapeDtypeStruct((1, 1), jnp.float32),
        scratch_shapes=[
            pltpu.VMEM((_P, _D), jnp.bfloat16),
            pltpu.VMEM((1, _P), jnp.float32),
            pltpu.VMEM((1, _P), jnp.float32),
            pltpu.VMEM((_BZ, 1), jnp.float32),
        ],
    )(z, prototype_vectors)
    return out[0, 0]
